# Initial kernel scaffold; baseline (speedup 1.0000x reference)
import jax
import jax.numpy as jnp
from jax.experimental import pallas as pl


def kernel(long_term_history, node_feats, conv1_w, conv1_b, conv2_w, conv2_b, bn1_g, bn1_b, bn2_g, bn2_b, bn3_g, bn3_b, fc_w, fc_b, fc_mean_w, fc_mean_b, fc_out_w, fc_out_b, fc_cat_w, fc_cat_b, tsf_w, tsf_b):
    raise NotImplementedError("write your pallas kernel here")


# speedup vs baseline: 1.2719x; 1.2719x over previous
import jax
import jax.numpy as jnp
from jax.experimental import pallas as pl

B = 4
N = 207
EMB = 100
K = 10
P = 168
L = 12
D_TSF = 96
NN = N * N
KK = K * N
RB = 8
NPAD = 208


def _bn_ncl(x, g, b, eps=1e-5):
    m = x.mean(axis=(0, 2), keepdims=True)
    v = x.var(axis=(0, 2), keepdims=True)
    return (x - m) / jnp.sqrt(v + eps) * g[None, :, None] + b[None, :, None]


def _bn_nd(x, g, b, eps=1e-5):
    m = x.mean(axis=0)
    v = x.var(axis=0)
    return (x - m) / jnp.sqrt(v + eps) * g + b


def _conv1d(x, w):
    return jax.lax.conv_general_dilated(x, w, (1,), 'VALID',
                                        dimension_numbers=('NCH', 'OIH', 'NCH'))


def _hs_kernel(x_ref, w_ref, b_ref, o_ref):
    o_ref[...] = jax.lax.dot_general(
        x_ref[...], w_ref[...], (((1,), (0,)), ((), ()))) + b_ref[...]


def _hidden_states(lth, tsf_w, tsf_b):
    x = lth[..., 0].reshape(B, P, L, N)
    x = jnp.transpose(x, (0, 3, 1, 2)).reshape(B * N * P, L)
    rows = B * N * P
    grid = 12
    blk = rows // grid
    out = pl.pallas_call(
        _hs_kernel,
        grid=(grid,),
        in_specs=[pl.BlockSpec((blk, L), lambda i: (i, 0)),
                  pl.BlockSpec((L, D_TSF), lambda i: (0, 0)),
                  pl.BlockSpec((1, D_TSF), lambda i: (0, 0))],
        out_specs=pl.BlockSpec((blk, D_TSF), lambda i: (i, 0)),
        out_shape=jax.ShapeDtypeStruct((rows, D_TSF), jnp.float32),
    )(x, tsf_w, tsf_b.reshape(1, D_TSF))
    return out.reshape(B, N, P, D_TSF)


def _edge_kernel(s_ref, r_ref, bo_ref, wc_ref, bc_ref, u_ref, log_ref, samp_ref):
    pid = pl.program_id(0)
    s = s_ref[...]
    cols = jax.lax.broadcasted_iota(jnp.int32, (1, N), 1)
    for i in range(RB):
        h = jax.nn.relu(s + r_ref[i:i + 1, :] + bo_ref[...])
        logits = jnp.dot(h, wc_ref[...]) + bc_ref[...]
        log_ref[i:i + 1, :, :] = logits.reshape(1, N, 2)
        l0 = logits[:, 0].reshape(1, N)
        l1 = logits[:, 1].reshape(1, N)
        offdiag = (cols != pid * RB + i).astype(jnp.float32)
        for b in range(B):
            u0 = u_ref[b, i:i + 1, :, 0]
            u1 = u_ref[b, i:i + 1, :, 1]
            g0 = -jnp.log(-jnp.log(u0 + 1e-20) + 1e-20)
            g1 = -jnp.log(-jnp.log(u1 + 1e-20) + 1e-20)
            z0 = (l0 + g0) * 2.0
            z1 = (l1 + g1) * 2.0
            m = jnp.maximum(z0, z1)
            e0 = jnp.exp(z0 - m)
            e1 = jnp.exp(z1 - m)
            y0 = e0 / (e0 + e1)
            hard = (z0 >= z1).astype(jnp.float32)
            samp_ref[b, i:i + 1, :] = ((hard - y0) + y0) * offdiag


def _sr_kernel(gf_ref, ws_ref, wr_ref, s_ref, r_ref):
    gf = gf_ref[...]
    s_ref[...] = jnp.dot(gf, ws_ref[...])
    r_ref[...] = jnp.dot(gf, wr_ref[...])


def _edge_and_sample(gf, fc_out_w, fc_out_b, fc_cat_w, fc_cat_b, u):
    s, r = pl.pallas_call(
        _sr_kernel,
        out_shape=(jax.ShapeDtypeStruct((N, EMB), jnp.float32),
                   jax.ShapeDtypeStruct((N, EMB), jnp.float32)),
    )(gf, fc_out_w[:EMB], fc_out_w[EMB:])
    r_pad = jnp.pad(r, ((0, NPAD - N), (0, 0)))
    u_pad = jnp.pad(u.reshape(B, N, N, 2), ((0, 0), (0, NPAD - N), (0, 0), (0, 0)))
    grid = NPAD // RB
    logits, samp = pl.pallas_call(
        _edge_kernel,
        grid=(grid,),
        in_specs=[pl.BlockSpec((N, EMB), lambda i: (0, 0)),
                  pl.BlockSpec((RB, EMB), lambda i: (i, 0)),
                  pl.BlockSpec((1, EMB), lambda i: (0, 0)),
                  pl.BlockSpec((EMB, 2), lambda i: (0, 0)),
                  pl.BlockSpec((1, 2), lambda i: (0, 0)),
                  pl.BlockSpec((B, RB, N, 2), lambda i: (0, i, 0, 0))],
        out_specs=(pl.BlockSpec((RB, N, 2), lambda i: (i, 0, 0)),
                   pl.BlockSpec((B, RB, N), lambda i: (0, i, 0))),
        out_shape=(jax.ShapeDtypeStruct((NPAD, N, 2), jnp.float32),
                   jax.ShapeDtypeStruct((B, NPAD, N), jnp.float32)),
    )(s, r_pad, fc_out_b.reshape(1, EMB), fc_cat_w, fc_cat_b.reshape(1, 2), u_pad)
    bern = jnp.broadcast_to(logits[:N].reshape(1, NN, 2), (B, NN, 2))
    sampled = samp[:, :N, :]
    return bern, sampled


def _knn_kernel(sim_ref, od_ref, adj_ref):
    flat = sim_ref[0]
    bits = jax.lax.bitcast_convert_type(flat, jnp.uint32)
    neg = (bits >> 31) != 0
    kv = jnp.where(neg, ~bits, bits | jnp.uint32(0x80000000))
    prefix = jnp.uint32(0)
    for bit in range(31, -1, -1):
        cand = prefix | jnp.uint32(1 << bit)
        himask = jnp.uint32((0xFFFFFFFF >> bit) << bit)
        cnt = jnp.sum(((kv & himask) >= cand).astype(jnp.int32))
        prefix = jnp.where(cnt >= KK, cand, prefix)
    kstar = prefix
    n_greater = jnp.sum((kv > kstar).astype(jnp.int32))
    t = KK - n_greater
    tie = kv == kstar
    idx = jax.lax.broadcasted_iota(jnp.int32, (1, NN), 1)
    p2 = jnp.int32(0)
    for bit in range(16, -1, -1):
        cand = p2 + jnp.int32(1 << bit)
        cnt = jnp.sum((tie & (idx < cand)).astype(jnp.int32))
        p2 = jnp.where(cnt < t, cand, p2)
    sel = (kv > kstar) | (tie & (idx <= p2) & (t > 0))
    keep = sel & (flat != 0.0) & (od_ref[0] != 0)
    adj_ref[0] = keep.astype(jnp.float32)


def _adj_knn(sim):
    flat = sim.reshape(B, 1, NN)
    offdiag = (1.0 - jnp.eye(N, dtype=jnp.float32)).reshape(1, NN)
    adj = pl.pallas_call(
        _knn_kernel,
        grid=(B,),
        in_specs=[pl.BlockSpec((1, 1, NN), lambda i: (i, 0, 0)),
                  pl.BlockSpec((1, NN), lambda i: (0, 0))],
        out_specs=pl.BlockSpec((1, 1, NN), lambda i: (i, 0, 0)),
        out_shape=jax.ShapeDtypeStruct((B, 1, NN), jnp.float32),
    )(flat, offdiag)
    return adj.reshape(B, N, N)


def kernel(long_term_history, node_feats, conv1_w, conv1_b, conv2_w, conv2_b,
           bn1_g, bn1_b, bn2_g, bn2_b, bn3_g, bn3_b, fc_w, fc_b,
           fc_mean_w, fc_mean_b, fc_out_w, fc_out_b, fc_cat_w, fc_cat_b,
           tsf_w, tsf_b):
    gf = node_feats.T.reshape(N, 1, -1)
    gf = _conv1d(gf, conv1_w) + conv1_b[None, :, None]
    gf = jax.nn.relu(gf)
    gf = _bn_ncl(gf, bn1_g, bn1_b)
    gf = _conv1d(gf, conv2_w) + conv2_b[None, :, None]
    gf = jax.nn.relu(gf)
    gf = _bn_ncl(gf, bn2_g, bn2_b)
    gf = gf.reshape(N, -1)
    gf = jax.nn.relu(gf @ fc_w + fc_b)
    gf = _bn_nd(gf, bn3_g, bn3_b)

    hidden_states = _hidden_states(long_term_history, tsf_w, tsf_b)

    u = jax.random.uniform(jax.random.key(42), (B, NN, 2), minval=0.0, maxval=1.0)
    bern, sampled_adj = _edge_and_sample(gf, fc_out_w, fc_out_b,
                                         fc_cat_w, fc_cat_b, u)

    d = hidden_states.reshape(B, N, -1)
    dn = d / (jnp.linalg.norm(d, axis=-1, keepdims=True) + 1e-10)
    sim = dn @ jnp.swapaxes(dn, -1, -2)

    adj_knn = _adj_knn(sim)

    return (bern, hidden_states, adj_knn, sampled_adj)


# speedup vs baseline: 1.3291x; 1.0450x over previous
import jax
import jax.numpy as jnp
from jax.experimental import pallas as pl

B = 4
N = 207
EMB = 100
K = 10
P = 168
L = 12
D_TSF = 96
NN = N * N
KK = K * N
RB = 8
NPAD = 208


def _bn_ncl(x, g, b, eps=1e-5):
    m = x.mean(axis=(0, 2), keepdims=True)
    v = x.var(axis=(0, 2), keepdims=True)
    return (x - m) / jnp.sqrt(v + eps) * g[None, :, None] + b[None, :, None]


def _bn_nd(x, g, b, eps=1e-5):
    m = x.mean(axis=0)
    v = x.var(axis=0)
    return (x - m) / jnp.sqrt(v + eps) * g + b


def _conv1d(x, w):
    return jax.lax.conv_general_dilated(x, w, (1,), 'VALID',
                                        dimension_numbers=('NCH', 'OIH', 'NCH'))


def _hs_kernel(x_ref, w_ref, b_ref, o_ref):
    o_ref[...] = jax.lax.dot_general(
        x_ref[...], w_ref[...], (((1,), (0,)), ((), ()))) + b_ref[...]


def _hidden_states(lth, tsf_w, tsf_b):
    x = lth[..., 0].reshape(B, P, L, N)
    x = jnp.transpose(x, (0, 3, 1, 2)).reshape(B * N * P, L)
    rows = B * N * P
    grid = 12
    blk = rows // grid
    out = pl.pallas_call(
        _hs_kernel,
        grid=(grid,),
        in_specs=[pl.BlockSpec((blk, L), lambda i: (i, 0)),
                  pl.BlockSpec((L, D_TSF), lambda i: (0, 0)),
                  pl.BlockSpec((1, D_TSF), lambda i: (0, 0))],
        out_specs=pl.BlockSpec((blk, D_TSF), lambda i: (i, 0)),
        out_shape=jax.ShapeDtypeStruct((rows, D_TSF), jnp.float32),
    )(x, tsf_w, tsf_b.reshape(1, D_TSF))
    return out.reshape(B, N, P, D_TSF)


def _edge_kernel(s_ref, r_ref, bo_ref, wc_ref, bc_ref, u_ref, log_ref, samp_ref):
    pid = pl.program_id(0)
    s = s_ref[...]
    cols = jax.lax.broadcasted_iota(jnp.int32, (1, N), 1)
    for i in range(RB):
        h = jax.nn.relu(s + r_ref[i:i + 1, :] + bo_ref[...])
        logits = jnp.dot(h, wc_ref[...]) + bc_ref[...]
        log_ref[i:i + 1, :, :] = logits.reshape(1, N, 2)
        l0 = logits[:, 0].reshape(1, N)
        l1 = logits[:, 1].reshape(1, N)
        offdiag = (cols != pid * RB + i).astype(jnp.float32)
        for b in range(B):
            u0 = u_ref[b, i:i + 1, :, 0]
            u1 = u_ref[b, i:i + 1, :, 1]
            g0 = -jnp.log(-jnp.log(u0 + 1e-20) + 1e-20)
            g1 = -jnp.log(-jnp.log(u1 + 1e-20) + 1e-20)
            z0 = (l0 + g0) * 2.0
            z1 = (l1 + g1) * 2.0
            m = jnp.maximum(z0, z1)
            e0 = jnp.exp(z0 - m)
            e1 = jnp.exp(z1 - m)
            y0 = e0 / (e0 + e1)
            hard = (z0 >= z1).astype(jnp.float32)
            samp_ref[b, i:i + 1, :] = ((hard - y0) + y0) * offdiag


def _sr_kernel(gf_ref, ws_ref, wr_ref, s_ref, r_ref):
    gf = gf_ref[...]
    s_ref[...] = jnp.dot(gf, ws_ref[...])
    r_ref[...] = jnp.dot(gf, wr_ref[...])


def _edge_and_sample(gf, fc_out_w, fc_out_b, fc_cat_w, fc_cat_b, u):
    s, r = pl.pallas_call(
        _sr_kernel,
        out_shape=(jax.ShapeDtypeStruct((N, EMB), jnp.float32),
                   jax.ShapeDtypeStruct((N, EMB), jnp.float32)),
    )(gf, fc_out_w[:EMB], fc_out_w[EMB:])
    r_pad = jnp.pad(r, ((0, NPAD - N), (0, 0)))
    u_pad = jnp.pad(u.reshape(B, N, N, 2), ((0, 0), (0, NPAD - N), (0, 0), (0, 0)))
    grid = NPAD // RB
    logits, samp = pl.pallas_call(
        _edge_kernel,
        grid=(grid,),
        in_specs=[pl.BlockSpec((N, EMB), lambda i: (0, 0)),
                  pl.BlockSpec((RB, EMB), lambda i: (i, 0)),
                  pl.BlockSpec((1, EMB), lambda i: (0, 0)),
                  pl.BlockSpec((EMB, 2), lambda i: (0, 0)),
                  pl.BlockSpec((1, 2), lambda i: (0, 0)),
                  pl.BlockSpec((B, RB, N, 2), lambda i: (0, i, 0, 0))],
        out_specs=(pl.BlockSpec((RB, N, 2), lambda i: (i, 0, 0)),
                   pl.BlockSpec((B, RB, N), lambda i: (0, i, 0))),
        out_shape=(jax.ShapeDtypeStruct((NPAD, N, 2), jnp.float32),
                   jax.ShapeDtypeStruct((B, NPAD, N), jnp.float32)),
    )(s, r_pad, fc_out_b.reshape(1, EMB), fc_cat_w, fc_cat_b.reshape(1, 2), u_pad)
    bern = jnp.broadcast_to(logits[:N].reshape(1, NN, 2), (B, NN, 2))
    sampled = samp[:, :N, :]
    return bern, sampled


def _knn_kernel(sim_ref, od_ref, adj_ref):
    flat = sim_ref[...]
    bits = jax.lax.bitcast_convert_type(flat, jnp.uint32)
    neg = (bits >> 31) != 0
    kv = jnp.where(neg, ~bits, bits | jnp.uint32(0x80000000))
    prefix = jnp.zeros((B, 1), jnp.uint32)
    for bit in range(31, -1, -1):
        cand = prefix | jnp.uint32(1 << bit)
        himask = jnp.uint32((0xFFFFFFFF >> bit) << bit)
        cnt = jnp.sum(((kv & himask) >= cand).astype(jnp.int32), axis=1, keepdims=True)
        prefix = jnp.where(cnt >= KK, cand, prefix)
    kstar = prefix
    n_greater = jnp.sum((kv > kstar).astype(jnp.int32), axis=1, keepdims=True)
    t = KK - n_greater
    tie = kv == kstar
    idx = jax.lax.broadcasted_iota(jnp.int32, (B, NN), 1)
    p2 = jnp.zeros((B, 1), jnp.int32)
    for bit in range(16, -1, -1):
        cand = p2 + jnp.int32(1 << bit)
        cnt = jnp.sum((tie & (idx < cand)).astype(jnp.int32), axis=1, keepdims=True)
        p2 = jnp.where(cnt < t, cand, p2)
    sel = (kv > kstar) | (tie & (idx <= p2) & (t > 0))
    keep = sel & (flat != 0.0) & (od_ref[...] != 0)
    adj_ref[...] = keep.astype(jnp.float32)


def _adj_knn(sim):
    flat = sim.reshape(B, NN)
    offdiag = (1.0 - jnp.eye(N, dtype=jnp.float32)).reshape(1, NN)
    adj = pl.pallas_call(
        _knn_kernel,
        in_specs=[pl.BlockSpec((B, NN), lambda: (0, 0)),
                  pl.BlockSpec((1, NN), lambda: (0, 0))],
        out_specs=pl.BlockSpec((B, NN), lambda: (0, 0)),
        out_shape=jax.ShapeDtypeStruct((B, NN), jnp.float32),
    )(flat, offdiag)
    return adj.reshape(B, N, N)


def kernel(long_term_history, node_feats, conv1_w, conv1_b, conv2_w, conv2_b,
           bn1_g, bn1_b, bn2_g, bn2_b, bn3_g, bn3_b, fc_w, fc_b,
           fc_mean_w, fc_mean_b, fc_out_w, fc_out_b, fc_cat_w, fc_cat_b,
           tsf_w, tsf_b):
    gf = node_feats.T.reshape(N, 1, -1)
    gf = _conv1d(gf, conv1_w) + conv1_b[None, :, None]
    gf = jax.nn.relu(gf)
    gf = _bn_ncl(gf, bn1_g, bn1_b)
    gf = _conv1d(gf, conv2_w) + conv2_b[None, :, None]
    gf = jax.nn.relu(gf)
    gf = _bn_ncl(gf, bn2_g, bn2_b)
    gf = gf.reshape(N, -1)
    gf = jax.nn.relu(gf @ fc_w + fc_b)
    gf = _bn_nd(gf, bn3_g, bn3_b)

    hidden_states = _hidden_states(long_term_history, tsf_w, tsf_b)

    u = jax.random.uniform(jax.random.key(42), (B, NN, 2), minval=0.0, maxval=1.0)
    bern, sampled_adj = _edge_and_sample(gf, fc_out_w, fc_out_b,
                                         fc_cat_w, fc_cat_b, u)

    d = hidden_states.reshape(B, N, -1)
    dn = d / (jnp.linalg.norm(d, axis=-1, keepdims=True) + 1e-10)
    sim = dn @ jnp.swapaxes(dn, -1, -2)

    adj_knn = _adj_knn(sim)

    return (bern, hidden_states, adj_knn, sampled_adj)


# speedup vs baseline: 1.3589x; 1.0224x over previous
import jax
import jax.numpy as jnp
from jax.experimental import pallas as pl

B = 4
N = 207
EMB = 100
K = 10
P = 168
L = 12
D_TSF = 96
NN = N * N
KK = K * N
RB = 8
NPAD = 208


def _bn_ncl(x, g, b, eps=1e-5):
    m = x.mean(axis=(0, 2), keepdims=True)
    v = x.var(axis=(0, 2), keepdims=True)
    return (x - m) / jnp.sqrt(v + eps) * g[None, :, None] + b[None, :, None]


def _bn_nd(x, g, b, eps=1e-5):
    m = x.mean(axis=0)
    v = x.var(axis=0)
    return (x - m) / jnp.sqrt(v + eps) * g + b


def _conv1d(x, w):
    return jax.lax.conv_general_dilated(x, w, (1,), 'VALID',
                                        dimension_numbers=('NCH', 'OIH', 'NCH'))


def _hs_kernel(x_ref, w_ref, b_ref, o_ref):
    o_ref[...] = jax.lax.dot_general(
        x_ref[...], w_ref[...], (((1,), (0,)), ((), ()))) + b_ref[...]


def _hidden_states(lth, tsf_w, tsf_b):
    x = lth[..., 0].reshape(B, P, L, N)
    x = jnp.transpose(x, (0, 3, 1, 2)).reshape(B * N * P, L)
    rows = B * N * P
    grid = 12
    blk = rows // grid
    out = pl.pallas_call(
        _hs_kernel,
        grid=(grid,),
        in_specs=[pl.BlockSpec((blk, L), lambda i: (i, 0)),
                  pl.BlockSpec((L, D_TSF), lambda i: (0, 0)),
                  pl.BlockSpec((1, D_TSF), lambda i: (0, 0))],
        out_specs=pl.BlockSpec((blk, D_TSF), lambda i: (i, 0)),
        out_shape=jax.ShapeDtypeStruct((rows, D_TSF), jnp.float32),
    )(x, tsf_w, tsf_b.reshape(1, D_TSF))
    return out.reshape(B, N, P, D_TSF)


def _edge_kernel(s_ref, r_ref, bo_ref, wc_ref, bc_ref, u_ref, log_ref, samp_ref):
    pid = pl.program_id(0)
    s = s_ref[...]
    cols = jax.lax.broadcasted_iota(jnp.int32, (1, N), 1)
    for i in range(RB):
        h = jax.nn.relu(s + r_ref[i:i + 1, :] + bo_ref[...])
        logits = jnp.dot(h, wc_ref[...]) + bc_ref[...]
        log_ref[i:i + 1, :, :] = logits.reshape(1, N, 2)
        l0 = logits[:, 0].reshape(1, N)
        l1 = logits[:, 1].reshape(1, N)
        offdiag = (cols != pid * RB + i).astype(jnp.float32)
        for b in range(B):
            u0 = u_ref[b, i:i + 1, :, 0]
            u1 = u_ref[b, i:i + 1, :, 1]
            g0 = -jnp.log(-jnp.log(u0 + 1e-20) + 1e-20)
            g1 = -jnp.log(-jnp.log(u1 + 1e-20) + 1e-20)
            z0 = (l0 + g0) * 2.0
            z1 = (l1 + g1) * 2.0
            m = jnp.maximum(z0, z1)
            e0 = jnp.exp(z0 - m)
            e1 = jnp.exp(z1 - m)
            y0 = e0 / (e0 + e1)
            hard = (z0 >= z1).astype(jnp.float32)
            samp_ref[b, i:i + 1, :] = ((hard - y0) + y0) * offdiag


def _sr_kernel(gf_ref, ws_ref, wr_ref, s_ref, r_ref):
    gf = gf_ref[...]
    s_ref[...] = jnp.dot(gf, ws_ref[...])
    r_ref[...] = jnp.dot(gf, wr_ref[...])


def _edge_and_sample(gf, fc_out_w, fc_out_b, fc_cat_w, fc_cat_b, u):
    s, r = pl.pallas_call(
        _sr_kernel,
        out_shape=(jax.ShapeDtypeStruct((N, EMB), jnp.float32),
                   jax.ShapeDtypeStruct((N, EMB), jnp.float32)),
    )(gf, fc_out_w[:EMB], fc_out_w[EMB:])
    r_pad = jnp.pad(r, ((0, NPAD - N), (0, 0)))
    u_pad = jnp.pad(u.reshape(B, N, N, 2), ((0, 0), (0, NPAD - N), (0, 0), (0, 0)))
    grid = NPAD // RB
    logits, samp = pl.pallas_call(
        _edge_kernel,
        grid=(grid,),
        in_specs=[pl.BlockSpec((N, EMB), lambda i: (0, 0)),
                  pl.BlockSpec((RB, EMB), lambda i: (i, 0)),
                  pl.BlockSpec((1, EMB), lambda i: (0, 0)),
                  pl.BlockSpec((EMB, 2), lambda i: (0, 0)),
                  pl.BlockSpec((1, 2), lambda i: (0, 0)),
                  pl.BlockSpec((B, RB, N, 2), lambda i: (0, i, 0, 0))],
        out_specs=(pl.BlockSpec((RB, N, 2), lambda i: (i, 0, 0)),
                   pl.BlockSpec((B, RB, N), lambda i: (0, i, 0))),
        out_shape=(jax.ShapeDtypeStruct((NPAD, N, 2), jnp.float32),
                   jax.ShapeDtypeStruct((B, NPAD, N), jnp.float32)),
    )(s, r_pad, fc_out_b.reshape(1, EMB), fc_cat_w, fc_cat_b.reshape(1, 2), u_pad)
    bern = jnp.broadcast_to(logits[:N].reshape(1, NN, 2), (B, NN, 2))
    sampled = samp[:, :N, :]
    return bern, sampled


def _knn_kernel(sim_ref, od_ref, adj_ref):
    flat = sim_ref[...]
    bits = jax.lax.bitcast_convert_type(flat, jnp.uint32)
    neg = (bits >> 31) != 0
    kv = jnp.where(neg, ~bits, bits | jnp.uint32(0x80000000))
    prefix = jnp.zeros((B, 1), jnp.uint32)
    for bit in range(31, -1, -1):
        cand = prefix | jnp.uint32(1 << bit)
        himask = jnp.uint32((0xFFFFFFFF >> bit) << bit)
        cnt = jnp.sum(((kv & himask) >= cand).astype(jnp.int32), axis=1, keepdims=True)
        prefix = jnp.where(cnt >= KK, cand, prefix)
    kstar = prefix
    n_greater = jnp.sum((kv > kstar).astype(jnp.int32), axis=1, keepdims=True)
    t = KK - n_greater
    tie = kv == kstar
    idx = jax.lax.broadcasted_iota(jnp.int32, (B, NN), 1)
    p2 = jnp.zeros((B, 1), jnp.int32)
    for bit in range(16, -1, -1):
        cand = p2 + jnp.int32(1 << bit)
        cnt = jnp.sum((tie & (idx < cand)).astype(jnp.int32), axis=1, keepdims=True)
        p2 = jnp.where(cnt < t, cand, p2)
    sel = (kv > kstar) | (tie & (idx <= p2) & (t > 0))
    keep = sel & (flat != 0.0) & (od_ref[...] != 0)
    adj_ref[...] = keep.astype(jnp.float32)


def _adj_knn(sim):
    flat = sim.reshape(B, NN)
    offdiag = (1.0 - jnp.eye(N, dtype=jnp.float32)).reshape(1, NN)
    adj = pl.pallas_call(
        _knn_kernel,
        in_specs=[pl.BlockSpec((B, NN), lambda: (0, 0)),
                  pl.BlockSpec((1, NN), lambda: (0, 0))],
        out_specs=pl.BlockSpec((B, NN), lambda: (0, 0)),
        out_shape=jax.ShapeDtypeStruct((B, NN), jnp.float32),
    )(flat, offdiag)
    return adj.reshape(B, N, N)


def kernel(long_term_history, node_feats, conv1_w, conv1_b, conv2_w, conv2_b,
           bn1_g, bn1_b, bn2_g, bn2_b, bn3_g, bn3_b, fc_w, fc_b,
           fc_mean_w, fc_mean_b, fc_out_w, fc_out_b, fc_cat_w, fc_cat_b,
           tsf_w, tsf_b):
    gf = node_feats.T.reshape(N, 1, -1)
    gf = _conv1d(gf, conv1_w) + conv1_b[None, :, None]
    gf = jax.nn.relu(gf)
    gf = _bn_ncl(gf, bn1_g, bn1_b)
    gf = _conv1d(gf, conv2_w) + conv2_b[None, :, None]
    gf = jax.nn.relu(gf)
    gf = _bn_ncl(gf, bn2_g, bn2_b)
    gf = gf.reshape(N, -1)
    gf = jax.nn.relu(gf @ fc_w + fc_b)
    gf = _bn_nd(gf, bn3_g, bn3_b)

    xx = long_term_history[..., 0].reshape(B, P, L, N)
    xx = jnp.transpose(xx, (0, 3, 1, 2))
    hidden_states = xx @ tsf_w + tsf_b

    u = jax.random.uniform(jax.random.key(42), (B, NN, 2), minval=0.0, maxval=1.0)
    bern, sampled_adj = _edge_and_sample(gf, fc_out_w, fc_out_b,
                                         fc_cat_w, fc_cat_b, u)

    d = hidden_states.reshape(B, N, -1)
    dn = d / (jnp.linalg.norm(d, axis=-1, keepdims=True) + 1e-10)
    sim = dn @ jnp.swapaxes(dn, -1, -2)

    adj_knn = _adj_knn(sim)

    return (bern, hidden_states, adj_knn, sampled_adj)
